# baseline (speedup 1.0000x reference)
import functools
import jax
import jax.numpy as jnp
import numpy as np
from jax import lax
from jax.experimental import pallas as pl
from jax.experimental.pallas import tpu as pltpu, tpu_sc as plsc

B, L, D = 4096, 50, 64
N = B * L
NC, NS, LANES = 2, 16, 16
NW = NC * NS
TOK = N // NW
CHUNK = 80
NCH = TOK // CHUNK
NP = NCH // 2
VPT = D // LANES

_mesh = plsc.VectorSubcoreMesh(core_axis_name="c", subcore_axis_name="s")

_scratch = (
    [pltpu.VMEM((NCH, CHUNK), jnp.int32) for _ in range(6)]
    + [pltpu.VMEM((CHUNK, D), jnp.float32) for _ in range(12)]
    + [pltpu.VMEM((CHUNK, D), jnp.float32) for _ in range(2)]
    + [pltpu.SemaphoreType.DMA for _ in range(4)]
)


@functools.partial(
    pl.kernel,
    out_type=jax.ShapeDtypeStruct((N, D), jnp.float32),
    mesh=_mesh,
    compiler_params=pltpu.CompilerParams(use_tc_tiling_on_sc=False),
    scratch_types=_scratch,
)
def _emb_sum(ip, ic, icol, isz, ig, Wp, Wc, Wcol, Ws, Wg, ctab, out, *sc):
    idxb = sc[0:5]
    codeb = sc[5]
    rows = (sc[6:12], sc[12:18])
    obuf = sc[18:20]
    gsem = sc[20:22]
    osem = sc[22:24]
    tabs = (Wp, Wc, Wcol, Ws, Wg)

    wid = lax.axis_index("s") * NC + lax.axis_index("c")
    wbase = wid * TOK

    for iref, vref in zip((ip, ic, icol, isz, ig), idxb):
        pltpu.sync_copy(iref.at[wid], vref)

    def code_body(c, _):
        for g in range(CHUNK // LANES):
            sl = pl.ds(g * LANES, LANES)
            code = jnp.where(idxb[0][c, sl] == 0, 1, 0)
            for t in range(1, 5):
                code = code + jnp.where(idxb[t][c, sl] == 0, 1 << t, 0)
            codeb[c, sl] = code
        return 0
    lax.fori_loop(0, NCH, code_body, 0, unroll=2)

    def g_copies(c, p):
        cps = [
            pltpu.make_async_copy(tab.at[ib.at[c]], rows[p][t], gsem[p])
            for t, (tab, ib) in enumerate(zip(tabs, idxb))
        ]
        cps.append(pltpu.make_async_copy(ctab.at[codeb.at[c]], rows[p][5], gsem[p]))
        return cps

    def o_copy(c, p):
        return pltpu.make_async_copy(
            obuf[p], out.at[pl.ds(wbase + c * CHUNK, CHUNK)], osem[p])

    def chunk_body(c, _):
        p = 0
        cps = g_copies(c, p)
        for cp in cps:
            cp.start()
        for cp in cps:
            cp.wait()

        r0, r1, r2, r3, r4, r5 = rows[p]
        ob = obuf[p]

        def add_body(i, _):
            for d in range(VPT):
                sl = pl.ds(d * LANES, LANES)
                ob[i, sl] = ((r0[i, sl] + r1[i, sl])
                             + (r2[i, sl] + r3[i, sl])
                             + (r4[i, sl] + r5[i, sl]))
            return 0
        lax.fori_loop(0, CHUNK, add_body, 0, unroll=2)

        pltpu.sync_copy(ob, out.at[pl.ds(wbase + c * CHUNK, CHUNK)])
        return 0

    lax.fori_loop(0, NCH, chunk_body, 0)


def kernel(product, customer, color, size, group,
           W_product, W_customer, W_color, W_size, W_group):
    bits = (np.arange(32)[:, None] >> np.arange(5)[None, :]) & 1
    w0 = jnp.stack([W_product[0], W_customer[0], W_color[0],
                    W_size[0], W_group[0]])
    ctab = -(jnp.asarray(bits, jnp.float32) @ w0)
    shp = (NW, NCH, CHUNK)
    out = _emb_sum(product.reshape(shp), customer.reshape(shp),
                   color.reshape(shp), size.reshape(shp), group.reshape(shp),
                   W_product, W_customer, W_color, W_size, W_group, ctab)
    return out.reshape(B, L, D)


# speedup vs baseline: 1.0007x; 1.0007x over previous
import functools
import jax
import jax.numpy as jnp
import numpy as np
from jax import lax
from jax.experimental import pallas as pl
from jax.experimental.pallas import tpu as pltpu, tpu_sc as plsc

B, L, D = 4096, 50, 64
N = B * L
NC, NS, LANES = 2, 16, 16
NW = NC * NS
TOK = N // NW
CHUNK = 80
NCH = TOK // CHUNK
NP = NCH // 2
VPT = D // LANES

_mesh = plsc.VectorSubcoreMesh(core_axis_name="c", subcore_axis_name="s")

_scratch = (
    [pltpu.VMEM((NCH, CHUNK), jnp.int32) for _ in range(6)]
    + [pltpu.VMEM((CHUNK, D), jnp.float32) for _ in range(12)]
    + [pltpu.VMEM((CHUNK, D), jnp.float32) for _ in range(2)]
    + [pltpu.SemaphoreType.DMA for _ in range(4)]
)


@functools.partial(
    pl.kernel,
    out_type=jax.ShapeDtypeStruct((N, D), jnp.float32),
    mesh=_mesh,
    compiler_params=pltpu.CompilerParams(use_tc_tiling_on_sc=False),
    scratch_types=_scratch,
)
def _emb_sum(ip, ic, icol, isz, ig, Wp, Wc, Wcol, Ws, Wg, ctab, out, *sc):
    idxb = sc[0:5]
    codeb = sc[5]
    rows = (sc[6:12], sc[12:18])
    obuf = sc[18:20]
    gsem = sc[20:22]
    osem = sc[22:24]
    tabs = (Wp, Wc, Wcol, Ws, Wg)

    wid = lax.axis_index("s") * NC + lax.axis_index("c")
    wbase = wid * TOK

    for iref, vref in zip((ip, ic, icol, isz, ig), idxb):
        pltpu.sync_copy(iref.at[wid], vref)

    def code_body(c, _):
        for g in range(CHUNK // LANES):
            sl = pl.ds(g * LANES, LANES)
            code = jnp.where(idxb[0][c, sl] == 0, 1, 0)
            for t in range(1, 5):
                code = code + jnp.where(idxb[t][c, sl] == 0, 1 << t, 0)
            codeb[c, sl] = code
        return 0
    lax.fori_loop(0, NCH, code_body, 0, unroll=2)

    def g_copies(c, p):
        cps = [
            pltpu.make_async_copy(tab.at[ib.at[c]], rows[p][t], gsem[p])
            for t, (tab, ib) in enumerate(zip(tabs, idxb))
        ]
        cps.append(pltpu.make_async_copy(ctab.at[codeb.at[c]], rows[p][5], gsem[p]))
        return cps

    def o_copy(c, p):
        return pltpu.make_async_copy(
            obuf[p], out.at[pl.ds(wbase + c * CHUNK, CHUNK)], osem[p])

    def chunk_body(c, _):
        p = 0
        cps = g_copies(c, p)
        for cp in cps:
            cp.start()
        for cp in cps:
            cp.wait()

        r0, r1, r2, r3, r4, r5 = rows[p]
        ob = obuf[p]

        @functools.partial(plsc.parallel_loop, 0, CHUNK, unroll=4)
        def _(i):
            for d in range(VPT):
                sl = pl.ds(d * LANES, LANES)
                ob[i, sl] = ((r0[i, sl] + r1[i, sl])
                             + (r2[i, sl] + r3[i, sl])
                             + (r4[i, sl] + r5[i, sl]))

        pltpu.sync_copy(ob, out.at[pl.ds(wbase + c * CHUNK, CHUNK)])
        return 0

    lax.fori_loop(0, NCH, chunk_body, 0)


def kernel(product, customer, color, size, group,
           W_product, W_customer, W_color, W_size, W_group):
    bits = (np.arange(32)[:, None] >> np.arange(5)[None, :]) & 1
    w0 = jnp.stack([W_product[0], W_customer[0], W_color[0],
                    W_size[0], W_group[0]])
    ctab = -(jnp.asarray(bits, jnp.float32) @ w0)
    shp = (NW, NCH, CHUNK)
    out = _emb_sum(product.reshape(shp), customer.reshape(shp),
                   color.reshape(shp), size.reshape(shp), group.reshape(shp),
                   W_product, W_customer, W_color, W_size, W_group, ctab)
    return out.reshape(B, L, D)


# speedup vs baseline: 1.0010x; 1.0003x over previous
import functools
import jax
import jax.numpy as jnp
import numpy as np
from jax import lax
from jax.experimental import pallas as pl
from jax.experimental.pallas import tpu as pltpu, tpu_sc as plsc

B, L, D = 4096, 50, 64
N = B * L
NC, NS, LANES = 2, 16, 16
NW = NC * NS
TOK = N // NW
CHUNK = 80
NCH = TOK // CHUNK
NP = NCH // 2
VPT = D // LANES

_mesh = plsc.VectorSubcoreMesh(core_axis_name="c", subcore_axis_name="s")

_scratch = (
    [pltpu.VMEM((NCH, CHUNK), jnp.int32) for _ in range(6)]
    + [pltpu.VMEM((CHUNK, D), jnp.float32) for _ in range(12)]
    + [pltpu.VMEM((CHUNK, D), jnp.float32) for _ in range(2)]
    + [pltpu.SemaphoreType.DMA for _ in range(14)]
)


@functools.partial(
    pl.kernel,
    out_type=jax.ShapeDtypeStruct((N, D), jnp.float32),
    mesh=_mesh,
    compiler_params=pltpu.CompilerParams(use_tc_tiling_on_sc=False),
    scratch_types=_scratch,
)
def _emb_sum(ip, ic, icol, isz, ig, Wp, Wc, Wcol, Ws, Wg, ctab, out, *sc):
    idxb = sc[0:5]
    codeb = sc[5]
    rows = (sc[6:12], sc[12:18])
    obuf = sc[18:20]
    gsem = sc[20:32]
    osem = sc[32:34]
    tabs = (Wp, Wc, Wcol, Ws, Wg)

    wid = lax.axis_index("s") * NC + lax.axis_index("c")
    wbase = wid * TOK

    for iref, vref in zip((ip, ic, icol, isz, ig), idxb):
        pltpu.sync_copy(iref.at[wid], vref)

    def code_body(c, _):
        for g in range(CHUNK // LANES):
            sl = pl.ds(g * LANES, LANES)
            code = jnp.where(idxb[0][c, sl] == 0, 1, 0)
            for t in range(1, 5):
                code = code + jnp.where(idxb[t][c, sl] == 0, 1 << t, 0)
            codeb[c, sl] = code
        return 0
    lax.fori_loop(0, NCH, code_body, 0, unroll=2)

    def g_copies(c, p):
        cps = [
            pltpu.make_async_copy(tab.at[ib.at[c]], rows[p][t], gsem[2 * t + p])
            for t, (tab, ib) in enumerate(zip(tabs, idxb))
        ]
        cps.append(pltpu.make_async_copy(ctab.at[codeb.at[c]], rows[p][5], gsem[10 + p]))
        return cps

    def o_copy(c, p):
        return pltpu.make_async_copy(
            obuf[p], out.at[pl.ds(wbase + c * CHUNK, CHUNK)], osem[p])

    def chunk_body(c, _):
        p = 0
        cps = g_copies(c, p)
        for cp in cps:
            cp.start()
        for cp in cps:
            cp.wait()

        r0, r1, r2, r3, r4, r5 = rows[p]
        ob = obuf[p]

        def add_body(i, _):
            for d in range(VPT):
                sl = pl.ds(d * LANES, LANES)
                ob[i, sl] = ((r0[i, sl] + r1[i, sl])
                             + (r2[i, sl] + r3[i, sl])
                             + (r4[i, sl] + r5[i, sl]))
            return 0
        lax.fori_loop(0, CHUNK, add_body, 0, unroll=2)

        pltpu.sync_copy(ob, out.at[pl.ds(wbase + c * CHUNK, CHUNK)])
        return 0

    lax.fori_loop(0, NCH, chunk_body, 0)


def kernel(product, customer, color, size, group,
           W_product, W_customer, W_color, W_size, W_group):
    bits = (np.arange(32)[:, None] >> np.arange(5)[None, :]) & 1
    w0 = jnp.stack([W_product[0], W_customer[0], W_color[0],
                    W_size[0], W_group[0]])
    ctab = -(jnp.asarray(bits, jnp.float32) @ w0)
    shp = (NW, NCH, CHUNK)
    out = _emb_sum(product.reshape(shp), customer.reshape(shp),
                   color.reshape(shp), size.reshape(shp), group.reshape(shp),
                   W_product, W_customer, W_color, W_size, W_group, ctab)
    return out.reshape(B, L, D)


# speedup vs baseline: 4.4273x; 4.4227x over previous
import functools
import jax
import jax.numpy as jnp
import numpy as np
from jax import lax
from jax.experimental import pallas as pl
from jax.experimental.pallas import tpu as pltpu, tpu_sc as plsc

B, L, D = 4096, 50, 64
N = B * L
NC, NS, LANES = 2, 16, 16
NW = NC * NS
TOK = N // NW
CHUNK = 80
NCH = TOK // CHUNK
NP = NCH // 2
VPT = D // LANES

_mesh = plsc.VectorSubcoreMesh(core_axis_name="c", subcore_axis_name="s")

_scratch = (
    [pltpu.VMEM((NCH, CHUNK), jnp.int32) for _ in range(6)]
    + [pltpu.VMEM((CHUNK, D), jnp.float32) for _ in range(12)]
    + [pltpu.VMEM((CHUNK, D), jnp.float32) for _ in range(2)]
    + [pltpu.SemaphoreType.DMA for _ in range(14)]
    + [pltpu.VMEM_SHARED((1000, D), jnp.float32) for _ in range(3)]
    + [pltpu.VMEM_SHARED((32, D), jnp.float32)]
)


@functools.partial(
    pl.kernel,
    out_type=jax.ShapeDtypeStruct((N, D), jnp.float32),
    mesh=_mesh,
    compiler_params=pltpu.CompilerParams(use_tc_tiling_on_sc=False),
    scratch_types=_scratch,
)
def _emb_sum(ip, ic, icol, isz, ig, Wp, Wc, Wcol, Ws, Wg, ctab, out, *sc):
    idxb = sc[0:5]
    codeb = sc[5]
    rows = (sc[6:12], sc[12:18])
    obuf = sc[18:20]
    gsem = sc[20:32]
    osem = sc[32:34]
    spm = sc[34:37]
    spm_ctab = sc[37]

    sid = lax.axis_index("s")
    wid = sid * NC + lax.axis_index("c")
    wbase = wid * TOK

    @pl.when(sid == 0)
    def _():
        for src, dst in zip((Wcol, Ws, Wg, ctab), (*spm, spm_ctab)):
            pltpu.sync_copy(src, dst)
    plsc.subcore_barrier()

    tabs = (Wp, Wc, spm[0], spm[1], spm[2])

    for iref, vref in zip((ip, ic, icol, isz, ig), idxb):
        pltpu.sync_copy(iref.at[wid], vref)

    def code_body(c, _):
        for g in range(CHUNK // LANES):
            sl = pl.ds(g * LANES, LANES)
            code = jnp.where(idxb[0][c, sl] == 0, 1, 0)
            for t in range(1, 5):
                code = code + jnp.where(idxb[t][c, sl] == 0, 1 << t, 0)
            codeb[c, sl] = code
        return 0
    lax.fori_loop(0, NCH, code_body, 0, unroll=2)

    def g_copies(c, p):
        cps = [
            pltpu.make_async_copy(tab.at[ib.at[c]], rows[p][t], gsem[2 * t + p])
            for t, (tab, ib) in enumerate(zip(tabs, idxb))
        ]
        cps.append(pltpu.make_async_copy(spm_ctab.at[codeb.at[c]], rows[p][5], gsem[10 + p]))
        return cps

    def o_copy(c, p):
        return pltpu.make_async_copy(
            obuf[p], out.at[pl.ds(wbase + c * CHUNK, CHUNK)], osem[p])

    def chunk_body(c, _):
        p = 0
        cps = g_copies(c, p)
        for cp in cps:
            cp.start()
        for cp in cps:
            cp.wait()

        r0, r1, r2, r3, r4, r5 = rows[p]
        ob = obuf[p]

        def add_body(i, _):
            for d in range(VPT):
                sl = pl.ds(d * LANES, LANES)
                ob[i, sl] = ((r0[i, sl] + r1[i, sl])
                             + (r2[i, sl] + r3[i, sl])
                             + (r4[i, sl] + r5[i, sl]))
            return 0
        lax.fori_loop(0, CHUNK, add_body, 0, unroll=2)

        pltpu.sync_copy(ob, out.at[pl.ds(wbase + c * CHUNK, CHUNK)])
        return 0

    lax.fori_loop(0, NCH, chunk_body, 0)


def kernel(product, customer, color, size, group,
           W_product, W_customer, W_color, W_size, W_group):
    bits = (np.arange(32)[:, None] >> np.arange(5)[None, :]) & 1
    w0 = jnp.stack([W_product[0], W_customer[0], W_color[0],
                    W_size[0], W_group[0]])
    ctab = -(jnp.asarray(bits, jnp.float32) @ w0)
    shp = (NW, NCH, CHUNK)
    out = _emb_sum(product.reshape(shp), customer.reshape(shp),
                   color.reshape(shp), size.reshape(shp), group.reshape(shp),
                   W_product, W_customer, W_color, W_size, W_group, ctab)
    return out.reshape(B, L, D)


# speedup vs baseline: 4.8508x; 1.0957x over previous
import functools
import jax
import jax.numpy as jnp
import numpy as np
from jax import lax
from jax.experimental import pallas as pl
from jax.experimental.pallas import tpu as pltpu, tpu_sc as plsc

B, L, D = 4096, 50, 64
N = B * L
NC, NS, LANES = 2, 16, 16
NW = NC * NS
TOK = N // NW
CHUNK = 80
NCH = TOK // CHUNK
NP = NCH // 2
VPT = D // LANES

_mesh = plsc.VectorSubcoreMesh(core_axis_name="c", subcore_axis_name="s")

_scratch = (
    [pltpu.VMEM((NCH, CHUNK), jnp.int32) for _ in range(6)]
    + [pltpu.VMEM((CHUNK, D), jnp.float32) for _ in range(12)]
    + [pltpu.VMEM((CHUNK, D), jnp.float32) for _ in range(2)]
    + [pltpu.SemaphoreType.DMA for _ in range(14)]
    + [pltpu.VMEM_SHARED((1000, D), jnp.float32) for _ in range(3)]
    + [pltpu.VMEM_SHARED((32, D), jnp.float32)]
)


@functools.partial(
    pl.kernel,
    out_type=jax.ShapeDtypeStruct((N, D), jnp.float32),
    mesh=_mesh,
    compiler_params=pltpu.CompilerParams(use_tc_tiling_on_sc=False),
    scratch_types=_scratch,
)
def _emb_sum(ip, ic, icol, isz, ig, Wp, Wc, Wcol, Ws, Wg, ctab, out, *sc):
    idxb = sc[0:5]
    codeb = sc[5]
    rows = (sc[6:12], sc[12:18])
    obuf = sc[18:20]
    gsem = sc[20:32]
    osem = sc[32:34]
    spm = sc[34:37]
    spm_ctab = sc[37]

    sid = lax.axis_index("s")
    wid = sid * NC + lax.axis_index("c")
    wbase = wid * TOK

    @pl.when(sid == 0)
    def _():
        for src, dst in zip((Wcol, Ws, Wg, ctab), (*spm, spm_ctab)):
            pltpu.sync_copy(src, dst)
    plsc.subcore_barrier()

    tabs = (Wp, Wc, spm[0], spm[1], spm[2])

    for iref, vref in zip((ip, ic, icol, isz, ig), idxb):
        pltpu.sync_copy(iref.at[wid], vref)

    def code_body(c, _):
        for g in range(CHUNK // LANES):
            sl = pl.ds(g * LANES, LANES)
            code = jnp.where(idxb[0][c, sl] == 0, 1, 0)
            for t in range(1, 5):
                code = code + jnp.where(idxb[t][c, sl] == 0, 1 << t, 0)
            codeb[c, sl] = code
        return 0
    lax.fori_loop(0, NCH, code_body, 0, unroll=2)

    def g_copies(c, p):
        cps = [
            pltpu.make_async_copy(tab.at[ib.at[c]], rows[p][t], gsem[2 * t + p])
            for t, (tab, ib) in enumerate(zip(tabs, idxb))
        ]
        cps.append(pltpu.make_async_copy(spm_ctab.at[codeb.at[c]], rows[p][5], gsem[10 + p]))
        return cps

    def o_copy(c, p):
        return pltpu.make_async_copy(
            obuf[p], out.at[pl.ds(wbase + c * CHUNK, CHUNK)], osem[p])

    for p in (0, 1):
        for cp in g_copies(jnp.int32(p), p):
            cp.start()

    def pair_body(cc, _):
        for p in (0, 1):
            c = cc * 2 + p
            for cp in g_copies(c, p):
                cp.wait()

            @pl.when(cc >= 1)
            def _():
                o_copy(c - 2, p).wait()

            r0, r1, r2, r3, r4, r5 = rows[p]
            ob = obuf[p]

            def add_body(i, _):
                for d in range(VPT):
                    sl = pl.ds(d * LANES, LANES)
                    ob[i, sl] = ((r0[i, sl] + r1[i, sl])
                                 + (r2[i, sl] + r3[i, sl])
                                 + (r4[i, sl] + r5[i, sl]))
                return 0
            lax.fori_loop(0, CHUNK, add_body, 0, unroll=2)

            o_copy(c, p).start()

            @pl.when(cc < NP - 1)
            def _():
                for cp in g_copies(c + 2, p):
                    cp.start()
        return 0

    lax.fori_loop(0, NP, pair_body, 0)
    for p in (0, 1):
        o_copy(jnp.int32((NP - 1) * 2 + p), p).wait()


def kernel(product, customer, color, size, group,
           W_product, W_customer, W_color, W_size, W_group):
    bits = (np.arange(32)[:, None] >> np.arange(5)[None, :]) & 1
    w0 = jnp.stack([W_product[0], W_customer[0], W_color[0],
                    W_size[0], W_group[0]])
    ctab = -(jnp.asarray(bits, jnp.float32) @ w0)
    shp = (NW, NCH, CHUNK)
    out = _emb_sum(product.reshape(shp), customer.reshape(shp),
                   color.reshape(shp), size.reshape(shp), group.reshape(shp),
                   W_product, W_customer, W_color, W_size, W_group, ctab)
    return out.reshape(B, L, D)
